# baseline (speedup 1.0000x reference)
import functools

import jax
import jax.numpy as jnp
from jax import lax
from jax.experimental import pallas as pl
from jax.experimental.pallas import tpu as pltpu
from jax.experimental.pallas import tpu_sc as plsc

_NC = 2
_NS = 16
_NW = _NC * _NS
_L = 16

_GC = 128
_H = 256


@jax.jit
def _run(data, Rr, Sr):
  B = data.shape[0] // 2
  W = Rr.shape[1]
  D = 32
  gpr = W // D
  bpw = B // _NW

  mesh = plsc.VectorSubcoreMesh(
      core_axis_name="c", subcore_axis_name="s",
      num_cores=_NC, num_subcores=_NS)

  @functools.partial(
      pl.kernel,
      out_type=jax.ShapeDtypeStruct((B,), jnp.float32),
      mesh=mesh,
      compiler_params=pltpu.CompilerParams(
          needs_layout_passes=False, use_tc_tiling_on_sc=True),
      scratch_types=[
          pltpu.VMEM((bpw * 2,), jnp.int32),
          pltpu.VMEM((bpw,), jnp.int32),
          pltpu.VMEM((bpw,), jnp.int32),
          pltpu.VMEM((bpw,), jnp.int32),
          pltpu.VMEM((bpw,), jnp.int32),
          pltpu.VMEM((_H, W), jnp.float32),
          pltpu.VMEM((_H, W), jnp.float32),
          pltpu.VMEM((bpw,), jnp.float32),
          pltpu.SemaphoreType.DMA,
      ],
  )
  def sc_kernel(data_hbm, r_hbm, s_hbm, out_hbm,
                dv, tv, uv, t4, u4, rv, sv, ov, sem):
    wid = lax.axis_index("s") * _NC + lax.axis_index("c")
    base = wid * bpw
    lane = lax.iota(jnp.int32, _L)

    pltpu.sync_copy(data_hbm.at[pl.ds(base * 2, bpw * 2)], dv)

    def deinterleave(b, carry):
      flat = (lane + b * _L) * 2
      off = pl.multiple_of(b * _L, _L)
      t = plsc.load_gather(dv, [flat])
      u = plsc.load_gather(dv, [flat + 1])
      tv[pl.ds(off, _L)] = t
      uv[pl.ds(off, _L)] = u
      t4[pl.ds(off, _L)] = lax.shift_right_logical(t, 2)
      u4[pl.ds(off, _L)] = lax.shift_right_logical(u, 2)
      return carry

    lax.fori_loop(0, bpw // _L, deinterleave, 0)

    for h in range(bpw // _H):
      copies = []
      for c in range(_H // _GC):
        off = h * _H + c * _GC
        copies.append(pltpu.async_copy(
            r_hbm.at[t4.at[pl.ds(off, _GC)]],
            rv.at[pl.ds(c * _GC, _GC), :], sem))
        copies.append(pltpu.async_copy(
            s_hbm.at[u4.at[pl.ds(off, _GC)]],
            sv.at[pl.ds(c * _GC, _GC), :], sem))
      for cp in copies:
        cp.wait()

      def block(b, carry):
        goff = pl.multiple_of(b * _L, _L)
        off = pl.multiple_of(h * _H + b * _L, _L)
        row = lane + goff
        tq = lax.mul(jnp.bitwise_and(tv[pl.ds(off, _L)], gpr - 1), D)
        uq = lax.mul(jnp.bitwise_and(uv[pl.ds(off, _L)], gpr - 1), D)
        acc = jnp.zeros((_L,), jnp.float32)
        for k in range(D):
          acc = acc + (plsc.load_gather(rv, [row, tq + k]) *
                       plsc.load_gather(sv, [row, uq + k]))
        ov[pl.ds(off, _L)] = acc
        return carry

      lax.fori_loop(0, _H // _L, block, 0)

    pltpu.sync_copy(ov, out_hbm.at[pl.ds(base, bpw)])

  return sc_kernel(data, Rr, Sr)


def kernel(data, R, S):
  n, d = R.shape
  return _run(data.reshape(-1), R.reshape(n // 4, d * 4), S.reshape(n // 4, d * 4))
